# baseline (speedup 1.0000x reference)
import functools

import jax
import jax.numpy as jnp
import numpy as np
from jax import lax
from jax.experimental import pallas as pl
from jax.experimental.pallas import tpu as pltpu
from jax.experimental.pallas import tpu_sc as plsc

_NC = 2
_NS = 16


def _node_precompute(x, w_src, Wp, Wq, cf):
    n, d = x.shape

    def body(x_ref, w_ref, wp_ref, wq_ref, cf_ref, tp_ref, tq_ref, u_ref):
        xv = x_ref[...]
        a = jnp.dot(xv, w_ref[...], preferred_element_type=jnp.float32)
        u = jnp.exp(a - jnp.max(a))
        alpha = jax.nn.sigmoid(cf_ref[...])
        p = jnp.dot(xv, wp_ref[...], preferred_element_type=jnp.float32)
        q = jnp.dot(xv, wq_ref[...], preferred_element_type=jnp.float32)
        tp_ref[...] = (alpha * u) * p
        tq_ref[...] = ((1.0 - alpha) * u) * q
        u_ref[...] = u

    return pl.pallas_call(
        body,
        out_shape=[
            jax.ShapeDtypeStruct((n, d), jnp.float32),
            jax.ShapeDtypeStruct((n, d), jnp.float32),
            jax.ShapeDtypeStruct((n, 1), jnp.float32),
        ],
    )(x, w_src, Wp, Wq, cf)


def _edge_pass(tpq, u, src, dst, rf, zrows, zs):
    n, d = tpq.shape
    e = src.shape[0]
    nt = _NC * _NS
    ept = e // nt
    ch = 48
    npairs = ept // (2 * ch)
    main = npairs * 2 * ch
    tail = ept - main
    hg = d // 16
    rpt = (n // _NS) & ~7
    rrem = n - _NS * rpt

    mesh = plsc.VectorSubcoreMesh(
        core_axis_name="c", subcore_axis_name="s",
        num_cores=_NC, num_subcores=_NS)

    @functools.partial(
        pl.kernel,
        mesh=mesh,
        compiler_params=pltpu.CompilerParams(needs_layout_passes=False),
        out_type=[
            jax.ShapeDtypeStruct((_NC, n, d), jnp.float32),
            jax.ShapeDtypeStruct((nt, 5, 1, n // 5), jnp.float32),
        ],
        scratch_types=[
            pltpu.VMEM((n,), jnp.float32),
            pltpu.VMEM((n,), jnp.float32),
            pltpu.VMEM((ch,), jnp.int32),
            pltpu.VMEM((ch,), jnp.int32),
            pltpu.VMEM((ch,), jnp.float32),
            pltpu.VMEM((ch,), jnp.int32),
            pltpu.VMEM((ch,), jnp.float32),
            pltpu.VMEM((ch,), jnp.int32),
            pltpu.VMEM((ch,), jnp.int32),
            pltpu.VMEM((ch,), jnp.float32),
            pltpu.VMEM((ch,), jnp.int32),
            pltpu.VMEM((ch,), jnp.float32),
            pltpu.VMEM((ch, d), jnp.int32),
            pltpu.VMEM((ch, d), jnp.int32),
            pltpu.VMEM((ch, d), jnp.float32),
            pltpu.VMEM((ch, d), jnp.float32),
            pltpu.VMEM((16,), jnp.int32),
            pltpu.VMEM((16,), jnp.int32),
            pltpu.VMEM((16,), jnp.float32),
            pltpu.VMEM_SHARED((n, d), jnp.float32),
            pltpu.SemaphoreType.DMA,
            pltpu.SemaphoreType.DMA,
            pltpu.SemaphoreType.DMA,
            pltpu.SemaphoreType.DMA,
            pltpu.SemaphoreType.DMA,
            pltpu.SemaphoreType.DMA,
        ],
    )
    def k(tpq_hbm, u_hbm, src_hbm, dst_hbm, rf_hbm, zrows_hbm, zs_hbm,
          acc_out, s_out,
          u_v, s_v,
          src_a, dst_a, rf_a, dsts_a, rfs_a,
          src_b, dst_b, rf_b, dsts_b, rfs_b,
          pq_a, pq_b, m_a, m_b,
          src_t, dst_t, rf_t,
          acc_sh,
          sem_ia, sem_ib, sem_pga, sem_pgb, sem_psa, sem_psb):
        c = lax.axis_index("c")
        s = lax.axis_index("s")
        wid = c * _NS + s
        tb = wid * ept

        @pl.when(s == 0)
        def _():
            pltpu.sync_copy(zrows_hbm, acc_sh)
        pltpu.sync_copy(zs_hbm, s_v)
        pltpu.sync_copy(u_hbm, u_v)
        plsc.subcore_barrier()

        zero16 = jnp.zeros((16,), jnp.int32)

        def s_channel(src_x, dst_x, nvec):
            for g in range(nvec):
                srcv = src_x[pl.ds(g * 16, 16)]
                dstv = dst_x[pl.ds(g * 16, 16)]
                uv = plsc.load_gather(u_v, [srcv])
                plsc.addupdate_scatter(s_v, [dstv], uv)

        def snapshot(from_x, to_x, nvec):
            for g in range(nvec):
                to_x[pl.ds(g * 16, 16)] = from_x[pl.ds(g * 16, 16)]

        himask = jnp.full((16,), -65536, jnp.int32)

        def scale(pq_x, m_x, rf_x, count):
            @plsc.parallel_loop(0, count, unroll=2)
            def _(ee):
                rfb = plsc.load_gather(rf_x, [zero16 + ee])
                for h in range(hg // 2):
                    pw = pq_x[ee, pl.ds(h * 16, 16)]
                    qw = pq_x[ee, pl.ds(d // 2 + h * 16, 16)]
                    plo = plsc.bitcast(lax.shift_left(pw, 16), jnp.float32)
                    phi = plsc.bitcast(pw & himask, jnp.float32)
                    qlo = plsc.bitcast(lax.shift_left(qw, 16), jnp.float32)
                    qhi = plsc.bitcast(qw & himask, jnp.float32)
                    m_x[ee, pl.ds(h * 32, 16)] = rfb * plo + qlo
                    m_x[ee, pl.ds(h * 32 + 16, 16)] = rfb * phi + qhi

        pltpu.sync_copy(src_hbm.at[pl.ds(tb + main, tail)], src_t)
        pltpu.sync_copy(dst_hbm.at[pl.ds(tb + main, tail)], dst_t)
        pltpu.sync_copy(rf_hbm.at[pl.ds(tb + main, tail)], rf_t)
        tg_cp = pltpu.async_copy(tpq_hbm.at[src_t], pq_a.at[pl.ds(0, tail)],
                                 sem_pga)
        s_channel(src_t, dst_t, tail // 16)
        tg_cp.wait()
        scale(pq_a, m_a, rf_t, tail)
        pltpu.sync_copy(m_a.at[pl.ds(0, tail)], acc_sh.at[dst_t], add=True)

        pltpu.sync_copy(src_hbm.at[pl.ds(tb, ch)], src_a)
        pltpu.sync_copy(dst_hbm.at[pl.ds(tb, ch)], dst_a)
        pltpu.sync_copy(rf_hbm.at[pl.ds(tb, ch)], rf_a)
        pltpu.async_copy(tpq_hbm.at[src_a], pq_a, sem_pga)
        pltpu.async_copy(src_hbm.at[pl.ds(tb + ch, ch)], src_b, sem_ib)
        pltpu.async_copy(dst_hbm.at[pl.ds(tb + ch, ch)], dst_b, sem_ib)
        pltpu.async_copy(rf_hbm.at[pl.ds(tb + ch, ch)], rf_b, sem_ib)

        def pair_body(j, carry):
            base = tb + j * (2 * ch)

            pltpu.make_async_copy(
                src_hbm.at[pl.ds(base + ch, ch)], src_b, sem_ib).wait()
            pltpu.make_async_copy(
                dst_hbm.at[pl.ds(base + ch, ch)], dst_b, sem_ib).wait()
            pltpu.make_async_copy(
                rf_hbm.at[pl.ds(base + ch, ch)], rf_b, sem_ib).wait()

            @pl.when(j > 0)
            def _():
                pltpu.make_async_copy(m_b, acc_sh.at[dsts_b], sem_psb).wait()

            pltpu.async_copy(tpq_hbm.at[src_b], pq_b, sem_pgb)

            s_channel(src_a, dst_a, ch // 16)
            pltpu.make_async_copy(tpq_hbm.at[src_a], pq_a, sem_pga).wait()
            snapshot(dst_a, dsts_a, ch // 16)
            snapshot(rf_a, rfs_a, ch // 16)

            @pl.when(j < npairs - 1)
            def _():
                nbase = base + 2 * ch
                pltpu.async_copy(src_hbm.at[pl.ds(nbase, ch)], src_a, sem_ia)
                pltpu.async_copy(dst_hbm.at[pl.ds(nbase, ch)], dst_a, sem_ia)
                pltpu.async_copy(rf_hbm.at[pl.ds(nbase, ch)], rf_a, sem_ia)

            scale(pq_a, m_a, rfs_a, ch)
            pltpu.async_copy(m_a, acc_sh.at[dsts_a], sem_psa, add=True)

            @pl.when(j < npairs - 1)
            def _():
                nbase = base + 2 * ch
                pltpu.make_async_copy(
                    src_hbm.at[pl.ds(nbase, ch)], src_a, sem_ia).wait()
                pltpu.make_async_copy(
                    dst_hbm.at[pl.ds(nbase, ch)], dst_a, sem_ia).wait()
                pltpu.make_async_copy(
                    rf_hbm.at[pl.ds(nbase, ch)], rf_a, sem_ia).wait()
                pltpu.make_async_copy(m_a, acc_sh.at[dsts_a], sem_psa).wait()
                pltpu.async_copy(tpq_hbm.at[src_a], pq_a, sem_pga)

            s_channel(src_b, dst_b, ch // 16)
            pltpu.make_async_copy(tpq_hbm.at[src_b], pq_b, sem_pgb).wait()
            snapshot(dst_b, dsts_b, ch // 16)
            snapshot(rf_b, rfs_b, ch // 16)

            @pl.when(j < npairs - 1)
            def _():
                nb2 = base + 3 * ch
                pltpu.async_copy(src_hbm.at[pl.ds(nb2, ch)], src_b, sem_ib)
                pltpu.async_copy(dst_hbm.at[pl.ds(nb2, ch)], dst_b, sem_ib)
                pltpu.async_copy(rf_hbm.at[pl.ds(nb2, ch)], rf_b, sem_ib)

            scale(pq_b, m_b, rfs_b, ch)
            pltpu.async_copy(m_b, acc_sh.at[dsts_b], sem_psb, add=True)
            return carry

        lax.fori_loop(0, npairs, pair_body, 0)

        pltpu.make_async_copy(m_a, acc_sh.at[dsts_a], sem_psa).wait()
        pltpu.make_async_copy(m_b, acc_sh.at[dsts_b], sem_psb).wait()
        plsc.subcore_barrier()

        r0 = s * rpt
        pltpu.sync_copy(acc_sh.at[pl.ds(r0, rpt)],
                        acc_out.at[c, pl.ds(r0, rpt)])

        @pl.when(s == _NS - 1)
        def _():
            pltpu.sync_copy(acc_sh.at[pl.ds(_NS * rpt, rrem)],
                            acc_out.at[c, pl.ds(_NS * rpt, rrem)])

        for i in range(5):
            pltpu.sync_copy(s_v.at[pl.ds(i * (n // 5), n // 5)],
                            s_out.at[wid, i, 0])

    return k(tpq, u, src, dst, rf, zrows, zs)


def _gelu(v):
    return 0.5 * v * (1.0 + lax.erf(v * 0.7071067811865476))


def _ln(v, g, b, eps=1e-5):
    mu = jnp.mean(v, axis=-1, keepdims=True)
    var = jnp.mean((v - mu) ** 2, axis=-1, keepdims=True)
    return (v - mu) / jnp.sqrt(var + eps) * g + b


def _finish(acc2, s32t, x, W1, b1, W2, b2, g1, beta1, g2, beta2):
    n, d = x.shape
    dh = W1.shape[1]
    br = 1000
    grid = n // br

    def body(acc_ref, s_ref, x_ref, w1_ref, b1_ref, w2_ref, b2_ref,
             g1_ref, be1_ref, g2_ref, be2_ref, o_ref):
        ssum = jnp.sum(s_ref[...], axis=1)
        acc = acc_ref[0] + acc_ref[1]
        recip = jnp.where(ssum > 0, 1.0 / ssum, 0.0)
        msg = acc * recip[:, None]
        y = _gelu(msg) + x_ref[...]
        o1 = _ln(y, g1_ref[...], be1_ref[...])
        h1 = _gelu(jnp.dot(o1, w1_ref[...],
                           preferred_element_type=jnp.float32) + b1_ref[...])
        h = jnp.dot(h1, w2_ref[...],
                    preferred_element_type=jnp.float32) + b2_ref[...]
        o_ref[...] = _ln(h + o1, g2_ref[...], be2_ref[...])

    return pl.pallas_call(
        body,
        grid=(grid,),
        in_specs=[
            pl.BlockSpec((2, br, d), lambda i: (0, i, 0)),
            pl.BlockSpec((br, _NC * _NS), lambda i: (i, 0)),
            pl.BlockSpec((br, d), lambda i: (i, 0)),
            pl.BlockSpec((d, dh), lambda i: (0, 0)),
            pl.BlockSpec((dh,), lambda i: (0,)),
            pl.BlockSpec((dh, d), lambda i: (0, 0)),
            pl.BlockSpec((d,), lambda i: (0,)),
            pl.BlockSpec((d,), lambda i: (0,)),
            pl.BlockSpec((d,), lambda i: (0,)),
            pl.BlockSpec((d,), lambda i: (0,)),
            pl.BlockSpec((d,), lambda i: (0,)),
        ],
        out_specs=pl.BlockSpec((br, d), lambda i: (i, 0)),
        out_shape=jax.ShapeDtypeStruct((n, d), jnp.float32),
    )(acc2, s32t, x, W1, b1, W2, b2, g1, beta1, g2, beta2)


def kernel(x, edge_index, W_phys, W_neur, att_w, channel_fusion, routing_factor,
           W1, b1, W2, b2, g1, beta1, g2, beta2):
    n, d = x.shape
    w_src = att_w[:d].reshape(d, 1)
    cf = jnp.asarray(channel_fusion, jnp.float32).reshape(1, 1)
    perm = np.empty((d,), np.int32)
    for h2 in range(d // 32):
        for i in range(16):
            for half in range(2):
                perm[h2 * 32 + 2 * i + half] = h2 * 32 + half * 16 + i
    tp, tq, u = _node_precompute(x, w_src, W_phys[:, perm], W_neur[:, perm],
                                 cf)

    def _pack(t):
        tb = t.astype(jnp.bfloat16).reshape(n, d // 2, 2)
        return lax.bitcast_convert_type(tb, jnp.int32)

    tpq = jnp.concatenate([_pack(tp), _pack(tq)], axis=1)
    src = edge_index[0]
    dst = edge_index[1]
    zrows = jnp.zeros((n, d), jnp.float32)
    zs = jnp.zeros((n,), jnp.float32)
    acc2, s32 = _edge_pass(tpq, u.reshape(n), src, dst, routing_factor,
                           zrows, zs)
    s32t = s32.reshape(_NC * _NS, n).T
    return _finish(acc2, s32t, x, W1, b1, W2, b2, g1, beta1, g2, beta2)


# speedup vs baseline: 1.2462x; 1.2462x over previous
import functools

import jax
import jax.numpy as jnp
import numpy as np
from jax import lax
from jax.experimental import pallas as pl
from jax.experimental.pallas import tpu as pltpu
from jax.experimental.pallas import tpu_sc as plsc

_NC = 2
_NS = 16


def _node_precompute(x, w_src, Wp, Wq, cf):
    n, d = x.shape

    def bf16_bits(f):
        ui = lax.bitcast_convert_type(f, jnp.int32)
        rnd = ui + 0x7FFF + (lax.shift_right_logical(ui, 16) & 1)
        return lax.shift_right_logical(rnd, 16)

    def pack_halves(t):
        lo = bf16_bits(t[:, :d // 2])
        hi = bf16_bits(t[:, d // 2:])
        return lo | lax.shift_left(hi, 16)

    def body(x_ref, w_ref, wp_ref, wq_ref, cf_ref, tpq_ref, u_ref):
        xv = x_ref[...]
        a = jnp.dot(xv, w_ref[...], preferred_element_type=jnp.float32)
        u = jnp.exp(a - jnp.max(a))
        alpha = jax.nn.sigmoid(cf_ref[...])
        p = jnp.dot(xv, wp_ref[...], preferred_element_type=jnp.float32)
        q = jnp.dot(xv, wq_ref[...], preferred_element_type=jnp.float32)
        tpq_ref[:, :d // 2] = pack_halves((alpha * u) * p)
        tpq_ref[:, d // 2:] = pack_halves(((1.0 - alpha) * u) * q)
        u_ref[...] = u

    return pl.pallas_call(
        body,
        out_shape=[
            jax.ShapeDtypeStruct((n, d), jnp.int32),
            jax.ShapeDtypeStruct((n, 1), jnp.float32),
        ],
    )(x, w_src, Wp, Wq, cf)


def _edge_pass(tpq, u, src, dst, rf, zrows, zs):
    n, d = tpq.shape
    e = src.shape[0]
    nt = _NC * _NS
    ept = e // nt
    ch = 48
    npairs = ept // (2 * ch)
    main = npairs * 2 * ch
    tail = ept - main
    hg = d // 16
    rpt = (n // _NS) & ~7
    rrem = n - _NS * rpt

    mesh = plsc.VectorSubcoreMesh(
        core_axis_name="c", subcore_axis_name="s",
        num_cores=_NC, num_subcores=_NS)

    @functools.partial(
        pl.kernel,
        mesh=mesh,
        compiler_params=pltpu.CompilerParams(needs_layout_passes=False),
        out_type=[
            jax.ShapeDtypeStruct((_NC, n, d), jnp.float32),
            jax.ShapeDtypeStruct((nt, 5, 1, n // 5), jnp.float32),
        ],
        scratch_types=[
            pltpu.VMEM((n,), jnp.float32),
            pltpu.VMEM((n,), jnp.float32),
            pltpu.VMEM((ch,), jnp.int32),
            pltpu.VMEM((ch,), jnp.int32),
            pltpu.VMEM((ch,), jnp.float32),
            pltpu.VMEM((ch,), jnp.int32),
            pltpu.VMEM((ch,), jnp.float32),
            pltpu.VMEM((ch,), jnp.int32),
            pltpu.VMEM((ch,), jnp.int32),
            pltpu.VMEM((ch,), jnp.float32),
            pltpu.VMEM((ch,), jnp.int32),
            pltpu.VMEM((ch,), jnp.float32),
            pltpu.VMEM((ch, d), jnp.int32),
            pltpu.VMEM((ch, d), jnp.int32),
            pltpu.VMEM((ch, d), jnp.float32),
            pltpu.VMEM((ch, d), jnp.float32),
            pltpu.VMEM((16,), jnp.int32),
            pltpu.VMEM((16,), jnp.int32),
            pltpu.VMEM((16,), jnp.float32),
            pltpu.VMEM_SHARED((n, d), jnp.float32),
            pltpu.SemaphoreType.DMA,
            pltpu.SemaphoreType.DMA,
            pltpu.SemaphoreType.DMA,
            pltpu.SemaphoreType.DMA,
            pltpu.SemaphoreType.DMA,
            pltpu.SemaphoreType.DMA,
        ],
    )
    def k(tpq_hbm, u_hbm, src_hbm, dst_hbm, rf_hbm, zrows_hbm, zs_hbm,
          acc_out, s_out,
          u_v, s_v,
          src_a, dst_a, rf_a, dsts_a, rfs_a,
          src_b, dst_b, rf_b, dsts_b, rfs_b,
          pq_a, pq_b, m_a, m_b,
          src_t, dst_t, rf_t,
          acc_sh,
          sem_ia, sem_ib, sem_pga, sem_pgb, sem_psa, sem_psb):
        c = lax.axis_index("c")
        s = lax.axis_index("s")
        wid = c * _NS + s
        tb = wid * ept

        @pl.when(s == 0)
        def _():
            pltpu.sync_copy(zrows_hbm, acc_sh)
        pltpu.sync_copy(zs_hbm, s_v)
        pltpu.sync_copy(u_hbm, u_v)
        plsc.subcore_barrier()

        zero16 = jnp.zeros((16,), jnp.int32)

        def s_channel(src_x, dst_x, nvec):
            for g in range(nvec):
                srcv = src_x[pl.ds(g * 16, 16)]
                dstv = dst_x[pl.ds(g * 16, 16)]
                uv = plsc.load_gather(u_v, [srcv])
                plsc.addupdate_scatter(s_v, [dstv], uv)

        def snapshot(from_x, to_x, nvec):
            for g in range(nvec):
                to_x[pl.ds(g * 16, 16)] = from_x[pl.ds(g * 16, 16)]

        himask = jnp.full((16,), -65536, jnp.int32)

        def scale(pq_x, m_x, rf_x, count):
            @plsc.parallel_loop(0, count, unroll=2)
            def _(ee):
                rfb = plsc.load_gather(rf_x, [zero16 + ee])
                for h in range(hg // 2):
                    pw = pq_x[ee, pl.ds(h * 16, 16)]
                    qw = pq_x[ee, pl.ds(d // 2 + h * 16, 16)]
                    plo = plsc.bitcast(lax.shift_left(pw, 16), jnp.float32)
                    phi = plsc.bitcast(pw & himask, jnp.float32)
                    qlo = plsc.bitcast(lax.shift_left(qw, 16), jnp.float32)
                    qhi = plsc.bitcast(qw & himask, jnp.float32)
                    m_x[ee, pl.ds(h * 32, 16)] = rfb * plo + qlo
                    m_x[ee, pl.ds(h * 32 + 16, 16)] = rfb * phi + qhi

        pltpu.sync_copy(src_hbm.at[pl.ds(tb + main, tail)], src_t)
        pltpu.sync_copy(dst_hbm.at[pl.ds(tb + main, tail)], dst_t)
        pltpu.sync_copy(rf_hbm.at[pl.ds(tb + main, tail)], rf_t)
        tg_cp = pltpu.async_copy(tpq_hbm.at[src_t], pq_a.at[pl.ds(0, tail)],
                                 sem_pga)
        s_channel(src_t, dst_t, tail // 16)
        tg_cp.wait()
        scale(pq_a, m_a, rf_t, tail)
        pltpu.sync_copy(m_a.at[pl.ds(0, tail)], acc_sh.at[dst_t], add=True)

        pltpu.sync_copy(src_hbm.at[pl.ds(tb, ch)], src_a)
        pltpu.sync_copy(dst_hbm.at[pl.ds(tb, ch)], dst_a)
        pltpu.sync_copy(rf_hbm.at[pl.ds(tb, ch)], rf_a)
        pltpu.async_copy(tpq_hbm.at[src_a], pq_a, sem_pga)
        pltpu.async_copy(src_hbm.at[pl.ds(tb + ch, ch)], src_b, sem_ib)
        pltpu.async_copy(dst_hbm.at[pl.ds(tb + ch, ch)], dst_b, sem_ib)
        pltpu.async_copy(rf_hbm.at[pl.ds(tb + ch, ch)], rf_b, sem_ib)

        def pair_body(j, carry):
            base = tb + j * (2 * ch)

            pltpu.make_async_copy(
                src_hbm.at[pl.ds(base + ch, ch)], src_b, sem_ib).wait()
            pltpu.make_async_copy(
                dst_hbm.at[pl.ds(base + ch, ch)], dst_b, sem_ib).wait()
            pltpu.make_async_copy(
                rf_hbm.at[pl.ds(base + ch, ch)], rf_b, sem_ib).wait()

            @pl.when(j > 0)
            def _():
                pltpu.make_async_copy(m_b, acc_sh.at[dsts_b], sem_psb).wait()

            pltpu.async_copy(tpq_hbm.at[src_b], pq_b, sem_pgb)

            s_channel(src_a, dst_a, ch // 16)
            pltpu.make_async_copy(tpq_hbm.at[src_a], pq_a, sem_pga).wait()
            snapshot(dst_a, dsts_a, ch // 16)
            snapshot(rf_a, rfs_a, ch // 16)

            @pl.when(j < npairs - 1)
            def _():
                nbase = base + 2 * ch
                pltpu.async_copy(src_hbm.at[pl.ds(nbase, ch)], src_a, sem_ia)
                pltpu.async_copy(dst_hbm.at[pl.ds(nbase, ch)], dst_a, sem_ia)
                pltpu.async_copy(rf_hbm.at[pl.ds(nbase, ch)], rf_a, sem_ia)

            scale(pq_a, m_a, rfs_a, ch)
            pltpu.async_copy(m_a, acc_sh.at[dsts_a], sem_psa, add=True)

            @pl.when(j < npairs - 1)
            def _():
                nbase = base + 2 * ch
                pltpu.make_async_copy(
                    src_hbm.at[pl.ds(nbase, ch)], src_a, sem_ia).wait()
                pltpu.make_async_copy(
                    dst_hbm.at[pl.ds(nbase, ch)], dst_a, sem_ia).wait()
                pltpu.make_async_copy(
                    rf_hbm.at[pl.ds(nbase, ch)], rf_a, sem_ia).wait()
                pltpu.make_async_copy(m_a, acc_sh.at[dsts_a], sem_psa).wait()
                pltpu.async_copy(tpq_hbm.at[src_a], pq_a, sem_pga)

            s_channel(src_b, dst_b, ch // 16)
            pltpu.make_async_copy(tpq_hbm.at[src_b], pq_b, sem_pgb).wait()
            snapshot(dst_b, dsts_b, ch // 16)
            snapshot(rf_b, rfs_b, ch // 16)

            @pl.when(j < npairs - 1)
            def _():
                nb2 = base + 3 * ch
                pltpu.async_copy(src_hbm.at[pl.ds(nb2, ch)], src_b, sem_ib)
                pltpu.async_copy(dst_hbm.at[pl.ds(nb2, ch)], dst_b, sem_ib)
                pltpu.async_copy(rf_hbm.at[pl.ds(nb2, ch)], rf_b, sem_ib)

            scale(pq_b, m_b, rfs_b, ch)
            pltpu.async_copy(m_b, acc_sh.at[dsts_b], sem_psb, add=True)
            return carry

        lax.fori_loop(0, npairs, pair_body, 0)

        pltpu.make_async_copy(m_a, acc_sh.at[dsts_a], sem_psa).wait()
        pltpu.make_async_copy(m_b, acc_sh.at[dsts_b], sem_psb).wait()
        plsc.subcore_barrier()

        r0 = s * rpt
        pltpu.sync_copy(acc_sh.at[pl.ds(r0, rpt)],
                        acc_out.at[c, pl.ds(r0, rpt)])

        @pl.when(s == _NS - 1)
        def _():
            pltpu.sync_copy(acc_sh.at[pl.ds(_NS * rpt, rrem)],
                            acc_out.at[c, pl.ds(_NS * rpt, rrem)])

        for i in range(5):
            pltpu.sync_copy(s_v.at[pl.ds(i * (n // 5), n // 5)],
                            s_out.at[wid, i, 0])

    return k(tpq, u, src, dst, rf, zrows, zs)


def _gelu(v):
    return 0.5 * v * (1.0 + lax.erf(v * 0.7071067811865476))


def _ln(v, g, b, eps=1e-5):
    mu = jnp.mean(v, axis=-1, keepdims=True)
    var = jnp.mean((v - mu) ** 2, axis=-1, keepdims=True)
    return (v - mu) / jnp.sqrt(var + eps) * g + b


def _finish(acc2, s32t, x, W1, b1, W2, b2, g1, beta1, g2, beta2):
    n, d = x.shape
    dh = W1.shape[1]
    br = 1000
    grid = n // br

    def body(acc_ref, s_ref, x_ref, w1_ref, b1_ref, w2_ref, b2_ref,
             g1_ref, be1_ref, g2_ref, be2_ref, o_ref):
        ssum = jnp.sum(s_ref[...], axis=1)
        acc = acc_ref[0] + acc_ref[1]
        recip = jnp.where(ssum > 0, 1.0 / ssum, 0.0)
        msg = acc * recip[:, None]
        y = _gelu(msg) + x_ref[...]
        o1 = _ln(y, g1_ref[...], be1_ref[...])
        h1 = _gelu(jnp.dot(o1, w1_ref[...],
                           preferred_element_type=jnp.float32) + b1_ref[...])
        h = jnp.dot(h1, w2_ref[...],
                    preferred_element_type=jnp.float32) + b2_ref[...]
        o_ref[...] = _ln(h + o1, g2_ref[...], be2_ref[...])

    return pl.pallas_call(
        body,
        grid=(grid,),
        in_specs=[
            pl.BlockSpec((2, br, d), lambda i: (0, i, 0)),
            pl.BlockSpec((br, _NC * _NS), lambda i: (i, 0)),
            pl.BlockSpec((br, d), lambda i: (i, 0)),
            pl.BlockSpec((d, dh), lambda i: (0, 0)),
            pl.BlockSpec((dh,), lambda i: (0,)),
            pl.BlockSpec((dh, d), lambda i: (0, 0)),
            pl.BlockSpec((d,), lambda i: (0,)),
            pl.BlockSpec((d,), lambda i: (0,)),
            pl.BlockSpec((d,), lambda i: (0,)),
            pl.BlockSpec((d,), lambda i: (0,)),
            pl.BlockSpec((d,), lambda i: (0,)),
        ],
        out_specs=pl.BlockSpec((br, d), lambda i: (i, 0)),
        out_shape=jax.ShapeDtypeStruct((n, d), jnp.float32),
    )(acc2, s32t, x, W1, b1, W2, b2, g1, beta1, g2, beta2)


def kernel(x, edge_index, W_phys, W_neur, att_w, channel_fusion, routing_factor,
           W1, b1, W2, b2, g1, beta1, g2, beta2):
    n, d = x.shape
    w_src = att_w[:d].reshape(d, 1)
    cf = jnp.asarray(channel_fusion, jnp.float32).reshape(1, 1)
    perm = np.empty((d,), np.int32)
    for p_ in range(d // 2):
        h2, i = divmod(p_, 16)
        perm[p_] = h2 * 32 + i
        perm[d // 2 + p_] = h2 * 32 + 16 + i
    tpq, u = _node_precompute(x, w_src, W_phys[:, perm], W_neur[:, perm], cf)
    src = edge_index[0]
    dst = edge_index[1]
    zrows = jnp.zeros((n, d), jnp.float32)
    zs = jnp.zeros((n,), jnp.float32)
    acc2, s32 = _edge_pass(tpq, u.reshape(n), src, dst, routing_factor,
                           zrows, zs)
    s32t = s32.reshape(_NC * _NS, n).T
    return _finish(acc2, s32t, x, W1, b1, W2, b2, g1, beta1, g2, beta2)


# speedup vs baseline: 1.2848x; 1.0309x over previous
import functools

import jax
import jax.numpy as jnp
import numpy as np
from jax import lax
from jax.experimental import pallas as pl
from jax.experimental.pallas import tpu as pltpu
from jax.experimental.pallas import tpu_sc as plsc

_NC = 2
_NS = 16


def _node_precompute(x, w_src, Wp, Wq, cf):
    n, d = x.shape

    def bf16_bits(f):
        ui = lax.bitcast_convert_type(f, jnp.int32)
        rnd = ui + 0x7FFF + (lax.shift_right_logical(ui, 16) & 1)
        return lax.shift_right_logical(rnd, 16)

    def pack_halves(t):
        lo = bf16_bits(t[:, :d // 2])
        hi = bf16_bits(t[:, d // 2:])
        return lo | lax.shift_left(hi, 16)

    def body(x_ref, w_ref, wp_ref, wq_ref, cf_ref, tpq_ref, u_ref):
        xv = x_ref[...]
        a = jnp.dot(xv, w_ref[...], preferred_element_type=jnp.float32)
        u = jnp.exp(a - jnp.max(a))
        alpha = jax.nn.sigmoid(cf_ref[...])
        p = jnp.dot(xv, wp_ref[...], preferred_element_type=jnp.float32)
        q = jnp.dot(xv, wq_ref[...], preferred_element_type=jnp.float32)
        tpq_ref[:, :d // 2] = pack_halves((alpha * u) * p)
        tpq_ref[:, d // 2:] = pack_halves(((1.0 - alpha) * u) * q)
        u_ref[...] = u

    return pl.pallas_call(
        body,
        out_shape=[
            jax.ShapeDtypeStruct((n, d), jnp.int32),
            jax.ShapeDtypeStruct((n, 1), jnp.float32),
        ],
    )(x, w_src, Wp, Wq, cf)


def _edge_pass(tpq, u, src, dst, rf, zrows, zs):
    n, d = tpq.shape
    e = src.shape[0]
    nt = _NC * _NS
    ept = e // nt
    ch = 48
    npairs = ept // (2 * ch)
    main = npairs * 2 * ch
    tail = ept - main
    hg = d // 16
    rpt = (n // _NS) & ~7
    rrem = n - _NS * rpt

    mesh = plsc.VectorSubcoreMesh(
        core_axis_name="c", subcore_axis_name="s",
        num_cores=_NC, num_subcores=_NS)

    @functools.partial(
        pl.kernel,
        mesh=mesh,
        compiler_params=pltpu.CompilerParams(needs_layout_passes=False),
        out_type=[
            jax.ShapeDtypeStruct((_NC, n, d), jnp.float32),
            jax.ShapeDtypeStruct((nt, 5, 1, n // 5), jnp.float32),
        ],
        scratch_types=[
            pltpu.VMEM((n,), jnp.float32),
            pltpu.VMEM((n,), jnp.float32),
            pltpu.VMEM((ch,), jnp.int32),
            pltpu.VMEM((ch,), jnp.int32),
            pltpu.VMEM((ch,), jnp.float32),
            pltpu.VMEM((ch,), jnp.int32),
            pltpu.VMEM((ch,), jnp.float32),
            pltpu.VMEM((ch,), jnp.int32),
            pltpu.VMEM((ch,), jnp.int32),
            pltpu.VMEM((ch,), jnp.float32),
            pltpu.VMEM((ch,), jnp.int32),
            pltpu.VMEM((ch,), jnp.float32),
            pltpu.VMEM((ch, d), jnp.int32),
            pltpu.VMEM((ch, d), jnp.int32),
            pltpu.VMEM((ch, d), jnp.float32),
            pltpu.VMEM((ch, d), jnp.float32),
            pltpu.VMEM((16,), jnp.int32),
            pltpu.VMEM((16,), jnp.int32),
            pltpu.VMEM((16,), jnp.float32),
            pltpu.VMEM_SHARED((n, d), jnp.float32),
            pltpu.SemaphoreType.DMA,
            pltpu.SemaphoreType.DMA,
            pltpu.SemaphoreType.DMA,
            pltpu.SemaphoreType.DMA,
            pltpu.SemaphoreType.DMA,
            pltpu.SemaphoreType.DMA,
        ],
    )
    def k(tpq_hbm, u_hbm, src_hbm, dst_hbm, rf_hbm, zrows_hbm, zs_hbm,
          acc_out, s_out,
          u_v, s_v,
          src_a, dst_a, rf_a, dsts_a, rfs_a,
          src_b, dst_b, rf_b, dsts_b, rfs_b,
          pq_a, pq_b, m_a, m_b,
          src_t, dst_t, rf_t,
          acc_sh,
          sem_ia, sem_ib, sem_pga, sem_pgb, sem_psa, sem_psb):
        c = lax.axis_index("c")
        s = lax.axis_index("s")
        wid = c * _NS + s
        tb = wid * ept

        @pl.when(s == 0)
        def _():
            pltpu.sync_copy(zrows_hbm, acc_sh)
        pltpu.sync_copy(zs_hbm, s_v)
        pltpu.sync_copy(u_hbm, u_v)
        plsc.subcore_barrier()

        zero16 = jnp.zeros((16,), jnp.int32)

        def s_channel(src_x, dst_x, nvec):
            for g in range(nvec):
                srcv = src_x[pl.ds(g * 16, 16)]
                dstv = dst_x[pl.ds(g * 16, 16)]
                uv = plsc.load_gather(u_v, [srcv])
                plsc.addupdate_scatter(s_v, [dstv], uv)

        def snapshot(from_x, to_x, nvec):
            for g in range(nvec):
                to_x[pl.ds(g * 16, 16)] = from_x[pl.ds(g * 16, 16)]

        himask = jnp.full((16,), -65536, jnp.int32)

        def scale(pq_x, m_x, rf_x, count):
            @plsc.parallel_loop(0, count, unroll=2)
            def _(ee):
                rfb = plsc.load_gather(rf_x, [zero16 + ee])
                for h in range(hg // 2):
                    pw = pq_x[ee, pl.ds(h * 16, 16)]
                    qw = pq_x[ee, pl.ds(d // 2 + h * 16, 16)]
                    plo = plsc.bitcast(lax.shift_left(pw, 16), jnp.float32)
                    phi = plsc.bitcast(pw & himask, jnp.float32)
                    qlo = plsc.bitcast(lax.shift_left(qw, 16), jnp.float32)
                    qhi = plsc.bitcast(qw & himask, jnp.float32)
                    m_x[ee, pl.ds(h * 32, 16)] = rfb * plo + qlo
                    m_x[ee, pl.ds(h * 32 + 16, 16)] = rfb * phi + qhi

        pltpu.sync_copy(src_hbm.at[pl.ds(tb + main, tail)], src_t)
        pltpu.sync_copy(dst_hbm.at[pl.ds(tb + main, tail)], dst_t)
        pltpu.sync_copy(rf_hbm.at[pl.ds(tb + main, tail)], rf_t)
        tg_cp = pltpu.async_copy(tpq_hbm.at[src_t], pq_a.at[pl.ds(0, tail)],
                                 sem_pga)
        s_channel(src_t, dst_t, tail // 16)
        tg_cp.wait()
        scale(pq_a, m_a, rf_t, tail)
        pltpu.sync_copy(m_a.at[pl.ds(0, tail)], acc_sh.at[dst_t], add=True)

        pltpu.sync_copy(src_hbm.at[pl.ds(tb, ch)], src_a)
        pltpu.sync_copy(dst_hbm.at[pl.ds(tb, ch)], dst_a)
        pltpu.sync_copy(rf_hbm.at[pl.ds(tb, ch)], rf_a)
        pltpu.async_copy(tpq_hbm.at[src_a], pq_a, sem_pga)
        pltpu.async_copy(src_hbm.at[pl.ds(tb + ch, ch)], src_b, sem_ib)
        pltpu.async_copy(dst_hbm.at[pl.ds(tb + ch, ch)], dst_b, sem_ib)
        pltpu.async_copy(rf_hbm.at[pl.ds(tb + ch, ch)], rf_b, sem_ib)

        def pair_body(j, carry):
            base = tb + j * (2 * ch)

            pltpu.make_async_copy(
                src_hbm.at[pl.ds(base + ch, ch)], src_b, sem_ib).wait()
            pltpu.make_async_copy(
                dst_hbm.at[pl.ds(base + ch, ch)], dst_b, sem_ib).wait()
            pltpu.make_async_copy(
                rf_hbm.at[pl.ds(base + ch, ch)], rf_b, sem_ib).wait()

            @pl.when(j > 0)
            def _():
                pltpu.make_async_copy(m_b, acc_sh.at[dsts_b], sem_psb).wait()

            pltpu.async_copy(tpq_hbm.at[src_b], pq_b, sem_pgb)

            s_channel(src_a, dst_a, ch // 16)
            pltpu.make_async_copy(tpq_hbm.at[src_a], pq_a, sem_pga).wait()
            snapshot(dst_a, dsts_a, ch // 16)
            snapshot(rf_a, rfs_a, ch // 16)

            @pl.when(j < npairs - 1)
            def _():
                nbase = base + 2 * ch
                pltpu.async_copy(src_hbm.at[pl.ds(nbase, ch)], src_a, sem_ia)
                pltpu.async_copy(dst_hbm.at[pl.ds(nbase, ch)], dst_a, sem_ia)
                pltpu.async_copy(rf_hbm.at[pl.ds(nbase, ch)], rf_a, sem_ia)

            scale(pq_a, m_a, rfs_a, ch)
            pltpu.async_copy(m_a, acc_sh.at[dsts_a], sem_psa, add=True)

            @pl.when(j < npairs - 1)
            def _():
                nbase = base + 2 * ch
                pltpu.make_async_copy(
                    src_hbm.at[pl.ds(nbase, ch)], src_a, sem_ia).wait()
                pltpu.make_async_copy(
                    dst_hbm.at[pl.ds(nbase, ch)], dst_a, sem_ia).wait()
                pltpu.make_async_copy(
                    rf_hbm.at[pl.ds(nbase, ch)], rf_a, sem_ia).wait()
                pltpu.make_async_copy(m_a, acc_sh.at[dsts_a], sem_psa).wait()
                pltpu.async_copy(tpq_hbm.at[src_a], pq_a, sem_pga)

            s_channel(src_b, dst_b, ch // 16)
            pltpu.make_async_copy(tpq_hbm.at[src_b], pq_b, sem_pgb).wait()
            snapshot(dst_b, dsts_b, ch // 16)
            snapshot(rf_b, rfs_b, ch // 16)

            @pl.when(j < npairs - 1)
            def _():
                nb2 = base + 3 * ch
                pltpu.async_copy(src_hbm.at[pl.ds(nb2, ch)], src_b, sem_ib)
                pltpu.async_copy(dst_hbm.at[pl.ds(nb2, ch)], dst_b, sem_ib)
                pltpu.async_copy(rf_hbm.at[pl.ds(nb2, ch)], rf_b, sem_ib)

            scale(pq_b, m_b, rfs_b, ch)
            pltpu.async_copy(m_b, acc_sh.at[dsts_b], sem_psb, add=True)
            return carry

        lax.fori_loop(0, npairs, pair_body, 0)

        pltpu.make_async_copy(m_a, acc_sh.at[dsts_a], sem_psa).wait()
        pltpu.make_async_copy(m_b, acc_sh.at[dsts_b], sem_psb).wait()
        plsc.subcore_barrier()

        r0 = s * rpt
        pltpu.sync_copy(acc_sh.at[pl.ds(r0, rpt)],
                        acc_out.at[c, pl.ds(r0, rpt)])

        @pl.when(s == _NS - 1)
        def _():
            pltpu.sync_copy(acc_sh.at[pl.ds(_NS * rpt, rrem)],
                            acc_out.at[c, pl.ds(_NS * rpt, rrem)])

        for i in range(5):
            pltpu.sync_copy(s_v.at[pl.ds(i * (n // 5), n // 5)],
                            s_out.at[wid, i, 0])

    return k(tpq, u, src, dst, rf, zrows, zs)


def _gelu(v):
    return 0.5 * v * (1.0 + lax.erf(v * 0.7071067811865476))


def _ln(v, g, b, eps=1e-5):
    mu = jnp.mean(v, axis=-1, keepdims=True)
    var = jnp.mean((v - mu) ** 2, axis=-1, keepdims=True)
    return (v - mu) / jnp.sqrt(var + eps) * g + b


def _finish(acc2, s32, x, W1, b1, W2, b2, g1, beta1, g2, beta2):
    n, d = x.shape
    dh = W1.shape[1]
    nt = s32.shape[0]
    br = n // 5
    grid = n // br

    def body(acc_ref, s_ref, x_ref, w1_ref, b1_ref, w2_ref, b2_ref,
             g1_ref, be1_ref, g2_ref, be2_ref, o_ref):
        ssum = jnp.sum(s_ref[...].reshape(nt, br), axis=0)
        acc = acc_ref[0] + acc_ref[1]
        recip = jnp.where(ssum > 0, 1.0 / ssum, 0.0)
        msg = acc * recip[:, None]
        y = _gelu(msg) + x_ref[...]
        o1 = _ln(y, g1_ref[...], be1_ref[...])
        h1 = _gelu(jnp.dot(o1, w1_ref[...],
                           preferred_element_type=jnp.float32) + b1_ref[...])
        h = jnp.dot(h1, w2_ref[...],
                    preferred_element_type=jnp.float32) + b2_ref[...]
        o_ref[...] = _ln(h + o1, g2_ref[...], be2_ref[...])

    return pl.pallas_call(
        body,
        grid=(grid,),
        in_specs=[
            pl.BlockSpec((2, br, d), lambda i: (0, i, 0)),
            pl.BlockSpec((nt, 1, 1, br), lambda i: (0, i, 0, 0)),
            pl.BlockSpec((br, d), lambda i: (i, 0)),
            pl.BlockSpec((d, dh), lambda i: (0, 0)),
            pl.BlockSpec((dh,), lambda i: (0,)),
            pl.BlockSpec((dh, d), lambda i: (0, 0)),
            pl.BlockSpec((d,), lambda i: (0,)),
            pl.BlockSpec((d,), lambda i: (0,)),
            pl.BlockSpec((d,), lambda i: (0,)),
            pl.BlockSpec((d,), lambda i: (0,)),
            pl.BlockSpec((d,), lambda i: (0,)),
        ],
        out_specs=pl.BlockSpec((br, d), lambda i: (i, 0)),
        out_shape=jax.ShapeDtypeStruct((n, d), jnp.float32),
    )(acc2, s32, x, W1, b1, W2, b2, g1, beta1, g2, beta2)


def kernel(x, edge_index, W_phys, W_neur, att_w, channel_fusion, routing_factor,
           W1, b1, W2, b2, g1, beta1, g2, beta2):
    n, d = x.shape
    w_src = att_w[:d].reshape(d, 1)
    cf = jnp.asarray(channel_fusion, jnp.float32).reshape(1, 1)
    perm = np.empty((d,), np.int32)
    for p_ in range(d // 2):
        h2, i = divmod(p_, 16)
        perm[p_] = h2 * 32 + i
        perm[d // 2 + p_] = h2 * 32 + 16 + i
    tpq, u = _node_precompute(x, w_src, W_phys[:, perm], W_neur[:, perm], cf)
    src = edge_index[0]
    dst = edge_index[1]
    zrows = jnp.zeros((n, d), jnp.float32)
    zs = jnp.zeros((n,), jnp.float32)
    acc2, s32 = _edge_pass(tpq, u.reshape(n), src, dst, routing_factor,
                           zrows, zs)
    return _finish(acc2, s32, x, W1, b1, W2, b2, g1, beta1, g2, beta2)


# speedup vs baseline: 1.4514x; 1.1297x over previous
import functools

import jax
import jax.numpy as jnp
import numpy as np
from jax import lax
from jax.experimental import pallas as pl
from jax.experimental.pallas import tpu as pltpu
from jax.experimental.pallas import tpu_sc as plsc

_NC = 2
_NS = 16


def _node_precompute(x, w_src, Wp, Wq, cf):
    n, d = x.shape

    def bf16_bits(f):
        ui = lax.bitcast_convert_type(f, jnp.int32)
        rnd = ui + 0x7FFF + (lax.shift_right_logical(ui, 16) & 1)
        return lax.shift_right_logical(rnd, 16)

    def pack_halves(t):
        lo = bf16_bits(t[:, :d // 2])
        hi = bf16_bits(t[:, d // 2:])
        return lo | lax.shift_left(hi, 16)

    def body(x_ref, w_ref, wp_ref, wq_ref, cf_ref, tpq_ref, u_ref):
        xv = x_ref[...]
        a = jnp.dot(xv, w_ref[...], preferred_element_type=jnp.float32)
        u = jnp.exp(a - jnp.max(a))
        alpha = jax.nn.sigmoid(cf_ref[...])
        p = jnp.dot(xv, wp_ref[...], preferred_element_type=jnp.float32)
        q = jnp.dot(xv, wq_ref[...], preferred_element_type=jnp.float32)
        tpq_ref[:, :d // 2] = pack_halves((alpha * u) * p)
        tpq_ref[:, d // 2:] = pack_halves(((1.0 - alpha) * u) * q)
        u_ref[...] = u

    return pl.pallas_call(
        body,
        out_shape=[
            jax.ShapeDtypeStruct((n, d), jnp.int32),
            jax.ShapeDtypeStruct((n, 1), jnp.float32),
        ],
    )(x, w_src, Wp, Wq, cf)


def _edge_pass(tpq, u, src, dst, rf, zrows, zs):
    n, d = tpq.shape
    e = src.shape[0]
    nt = _NC * _NS
    ept = e // nt
    ch = 64
    npairs = ept // (2 * ch)
    main = npairs * 2 * ch
    tail = ept - main
    hg = d // 16
    rpt = (n // _NS) & ~7
    rrem = n - _NS * rpt

    mesh = plsc.VectorSubcoreMesh(
        core_axis_name="c", subcore_axis_name="s",
        num_cores=_NC, num_subcores=_NS)

    @functools.partial(
        pl.kernel,
        mesh=mesh,
        compiler_params=pltpu.CompilerParams(needs_layout_passes=False),
        out_type=[
            jax.ShapeDtypeStruct((_NC, n, d), jnp.float32),
            jax.ShapeDtypeStruct((nt, 5, 1, n // 5), jnp.float32),
        ],
        scratch_types=[
            pltpu.VMEM((n,), jnp.float32),
            pltpu.VMEM((n,), jnp.float32),
            pltpu.VMEM((ch,), jnp.int32),
            pltpu.VMEM((ch,), jnp.int32),
            pltpu.VMEM((ch,), jnp.float32),
            pltpu.VMEM((ch,), jnp.int32),
            pltpu.VMEM((ch,), jnp.float32),
            pltpu.VMEM((ch,), jnp.int32),
            pltpu.VMEM((ch,), jnp.int32),
            pltpu.VMEM((ch,), jnp.float32),
            pltpu.VMEM((ch,), jnp.int32),
            pltpu.VMEM((ch,), jnp.float32),
            pltpu.VMEM((ch, d), jnp.int32),
            pltpu.VMEM((ch, d), jnp.int32),
            pltpu.VMEM((ch, d), jnp.float32),
            pltpu.VMEM((16,), jnp.int32),
            pltpu.VMEM((16,), jnp.int32),
            pltpu.VMEM((16,), jnp.float32),
            pltpu.VMEM_SHARED((n, d), jnp.float32),
            pltpu.SemaphoreType.DMA,
            pltpu.SemaphoreType.DMA,
            pltpu.SemaphoreType.DMA,
            pltpu.SemaphoreType.DMA,
            pltpu.SemaphoreType.DMA,
        ],
    )
    def k(tpq_hbm, u_hbm, src_hbm, dst_hbm, rf_hbm, zrows_hbm, zs_hbm,
          acc_out, s_out,
          u_v, s_v,
          src_a, dst_a, rf_a, dsts_a, rfs_a,
          src_b, dst_b, rf_b, dsts_b, rfs_b,
          pq_a, pq_b, m,
          src_t, dst_t, rf_t,
          acc_sh,
          sem_ia, sem_ib, sem_pga, sem_pgb, sem_ps):
        c = lax.axis_index("c")
        s = lax.axis_index("s")
        wid = c * _NS + s
        tb = wid * ept

        @pl.when(s == 0)
        def _():
            pltpu.sync_copy(zrows_hbm, acc_sh)
        pltpu.sync_copy(zs_hbm, s_v)
        pltpu.sync_copy(u_hbm, u_v)
        plsc.subcore_barrier()

        zero16 = jnp.zeros((16,), jnp.int32)

        def s_channel(src_x, dst_x, nvec):
            for g in range(nvec):
                srcv = src_x[pl.ds(g * 16, 16)]
                dstv = dst_x[pl.ds(g * 16, 16)]
                uv = plsc.load_gather(u_v, [srcv])
                plsc.addupdate_scatter(s_v, [dstv], uv)

        def snapshot(from_x, to_x, nvec):
            for g in range(nvec):
                to_x[pl.ds(g * 16, 16)] = from_x[pl.ds(g * 16, 16)]

        himask = jnp.full((16,), -65536, jnp.int32)

        def scale(pq_x, m_x, rf_x, count):
            @plsc.parallel_loop(0, count, unroll=2)
            def _(ee):
                rfb = plsc.load_gather(rf_x, [zero16 + ee])
                for h in range(hg // 2):
                    pw = pq_x[ee, pl.ds(h * 16, 16)]
                    qw = pq_x[ee, pl.ds(d // 2 + h * 16, 16)]
                    plo = plsc.bitcast(lax.shift_left(pw, 16), jnp.float32)
                    phi = plsc.bitcast(pw & himask, jnp.float32)
                    qlo = plsc.bitcast(lax.shift_left(qw, 16), jnp.float32)
                    qhi = plsc.bitcast(qw & himask, jnp.float32)
                    m_x[ee, pl.ds(h * 32, 16)] = rfb * plo + qlo
                    m_x[ee, pl.ds(h * 32 + 16, 16)] = rfb * phi + qhi

        pltpu.sync_copy(src_hbm.at[pl.ds(tb + main, tail)], src_t)
        pltpu.sync_copy(dst_hbm.at[pl.ds(tb + main, tail)], dst_t)
        pltpu.sync_copy(rf_hbm.at[pl.ds(tb + main, tail)], rf_t)
        tg_cp = pltpu.async_copy(tpq_hbm.at[src_t], pq_a.at[pl.ds(0, tail)],
                                 sem_pga)
        s_channel(src_t, dst_t, tail // 16)
        tg_cp.wait()
        scale(pq_a, m, rf_t, tail)
        pltpu.sync_copy(m.at[pl.ds(0, tail)], acc_sh.at[dst_t], add=True)

        pltpu.sync_copy(src_hbm.at[pl.ds(tb, ch)], src_a)
        pltpu.sync_copy(dst_hbm.at[pl.ds(tb, ch)], dst_a)
        pltpu.sync_copy(rf_hbm.at[pl.ds(tb, ch)], rf_a)
        pltpu.async_copy(tpq_hbm.at[src_a], pq_a, sem_pga)
        pltpu.async_copy(src_hbm.at[pl.ds(tb + ch, ch)], src_b, sem_ib)
        pltpu.async_copy(dst_hbm.at[pl.ds(tb + ch, ch)], dst_b, sem_ib)
        pltpu.async_copy(rf_hbm.at[pl.ds(tb + ch, ch)], rf_b, sem_ib)

        def pair_body(j, carry):
            base = tb + j * (2 * ch)

            pltpu.make_async_copy(
                src_hbm.at[pl.ds(base + ch, ch)], src_b, sem_ib).wait()
            pltpu.make_async_copy(
                dst_hbm.at[pl.ds(base + ch, ch)], dst_b, sem_ib).wait()
            pltpu.make_async_copy(
                rf_hbm.at[pl.ds(base + ch, ch)], rf_b, sem_ib).wait()

            pltpu.async_copy(tpq_hbm.at[src_b], pq_b, sem_pgb)

            s_channel(src_a, dst_a, ch // 16)
            pltpu.make_async_copy(tpq_hbm.at[src_a], pq_a, sem_pga).wait()
            snapshot(dst_a, dsts_a, ch // 16)
            snapshot(rf_a, rfs_a, ch // 16)

            @pl.when(j < npairs - 1)
            def _():
                nbase = base + 2 * ch
                pltpu.async_copy(src_hbm.at[pl.ds(nbase, ch)], src_a, sem_ia)
                pltpu.async_copy(dst_hbm.at[pl.ds(nbase, ch)], dst_a, sem_ia)
                pltpu.async_copy(rf_hbm.at[pl.ds(nbase, ch)], rf_a, sem_ia)

            @pl.when(j > 0)
            def _():
                pltpu.make_async_copy(m, acc_sh.at[dsts_b], sem_ps).wait()

            scale(pq_a, m, rfs_a, ch)
            pltpu.async_copy(m, acc_sh.at[dsts_a], sem_ps, add=True)

            @pl.when(j < npairs - 1)
            def _():
                nbase = base + 2 * ch
                pltpu.make_async_copy(
                    src_hbm.at[pl.ds(nbase, ch)], src_a, sem_ia).wait()
                pltpu.make_async_copy(
                    dst_hbm.at[pl.ds(nbase, ch)], dst_a, sem_ia).wait()
                pltpu.make_async_copy(
                    rf_hbm.at[pl.ds(nbase, ch)], rf_a, sem_ia).wait()
                pltpu.async_copy(tpq_hbm.at[src_a], pq_a, sem_pga)

            s_channel(src_b, dst_b, ch // 16)
            pltpu.make_async_copy(tpq_hbm.at[src_b], pq_b, sem_pgb).wait()
            snapshot(dst_b, dsts_b, ch // 16)
            snapshot(rf_b, rfs_b, ch // 16)

            @pl.when(j < npairs - 1)
            def _():
                nb2 = base + 3 * ch
                pltpu.async_copy(src_hbm.at[pl.ds(nb2, ch)], src_b, sem_ib)
                pltpu.async_copy(dst_hbm.at[pl.ds(nb2, ch)], dst_b, sem_ib)
                pltpu.async_copy(rf_hbm.at[pl.ds(nb2, ch)], rf_b, sem_ib)

            pltpu.make_async_copy(m, acc_sh.at[dsts_a], sem_ps).wait()
            scale(pq_b, m, rfs_b, ch)
            pltpu.async_copy(m, acc_sh.at[dsts_b], sem_ps, add=True)
            return carry

        lax.fori_loop(0, npairs, pair_body, 0)

        pltpu.make_async_copy(m, acc_sh.at[dsts_b], sem_ps).wait()
        plsc.subcore_barrier()

        r0 = s * rpt
        pltpu.sync_copy(acc_sh.at[pl.ds(r0, rpt)],
                        acc_out.at[c, pl.ds(r0, rpt)])

        @pl.when(s == _NS - 1)
        def _():
            pltpu.sync_copy(acc_sh.at[pl.ds(_NS * rpt, rrem)],
                            acc_out.at[c, pl.ds(_NS * rpt, rrem)])

        for i in range(5):
            pltpu.sync_copy(s_v.at[pl.ds(i * (n // 5), n // 5)],
                            s_out.at[wid, i, 0])

    return k(tpq, u, src, dst, rf, zrows, zs)


def _gelu(v):
    return 0.5 * v * (1.0 + lax.erf(v * 0.7071067811865476))


def _ln(v, g, b, eps=1e-5):
    mu = jnp.mean(v, axis=-1, keepdims=True)
    var = jnp.mean((v - mu) ** 2, axis=-1, keepdims=True)
    return (v - mu) / jnp.sqrt(var + eps) * g + b


def _finish(acc2, s32, x, W1, b1, W2, b2, g1, beta1, g2, beta2):
    n, d = x.shape
    dh = W1.shape[1]
    nt = s32.shape[0]
    br = n // 5
    grid = n // br

    def body(acc_ref, s_ref, x_ref, w1_ref, b1_ref, w2_ref, b2_ref,
             g1_ref, be1_ref, g2_ref, be2_ref, o_ref):
        ssum = jnp.sum(s_ref[...].reshape(nt, br), axis=0)
        acc = acc_ref[0] + acc_ref[1]
        recip = jnp.where(ssum > 0, 1.0 / ssum, 0.0)
        msg = acc * recip[:, None]
        y = _gelu(msg) + x_ref[...]
        o1 = _ln(y, g1_ref[...], be1_ref[...])
        h1 = _gelu(jnp.dot(o1, w1_ref[...],
                           preferred_element_type=jnp.float32) + b1_ref[...])
        h = jnp.dot(h1, w2_ref[...],
                    preferred_element_type=jnp.float32) + b2_ref[...]
        o_ref[...] = _ln(h + o1, g2_ref[...], be2_ref[...])

    return pl.pallas_call(
        body,
        grid=(grid,),
        in_specs=[
            pl.BlockSpec((2, br, d), lambda i: (0, i, 0)),
            pl.BlockSpec((nt, 1, 1, br), lambda i: (0, i, 0, 0)),
            pl.BlockSpec((br, d), lambda i: (i, 0)),
            pl.BlockSpec((d, dh), lambda i: (0, 0)),
            pl.BlockSpec((dh,), lambda i: (0,)),
            pl.BlockSpec((dh, d), lambda i: (0, 0)),
            pl.BlockSpec((d,), lambda i: (0,)),
            pl.BlockSpec((d,), lambda i: (0,)),
            pl.BlockSpec((d,), lambda i: (0,)),
            pl.BlockSpec((d,), lambda i: (0,)),
            pl.BlockSpec((d,), lambda i: (0,)),
        ],
        out_specs=pl.BlockSpec((br, d), lambda i: (i, 0)),
        out_shape=jax.ShapeDtypeStruct((n, d), jnp.float32),
    )(acc2, s32, x, W1, b1, W2, b2, g1, beta1, g2, beta2)


def kernel(x, edge_index, W_phys, W_neur, att_w, channel_fusion, routing_factor,
           W1, b1, W2, b2, g1, beta1, g2, beta2):
    n, d = x.shape
    w_src = att_w[:d].reshape(d, 1)
    cf = jnp.asarray(channel_fusion, jnp.float32).reshape(1, 1)
    perm = np.empty((d,), np.int32)
    for p_ in range(d // 2):
        h2, i = divmod(p_, 16)
        perm[p_] = h2 * 32 + i
        perm[d // 2 + p_] = h2 * 32 + 16 + i
    tpq, u = _node_precompute(x, w_src, W_phys[:, perm], W_neur[:, perm], cf)
    src = edge_index[0]
    dst = edge_index[1]
    zrows = jnp.zeros((n, d), jnp.float32)
    zs = jnp.zeros((n,), jnp.float32)
    acc2, s32 = _edge_pass(tpq, u.reshape(n), src, dst, routing_factor,
                           zrows, zs)
    return _finish(acc2, s32, x, W1, b1, W2, b2, g1, beta1, g2, beta2)


# speedup vs baseline: 1.4529x; 1.0010x over previous
import functools

import jax
import jax.numpy as jnp
import numpy as np
from jax import lax
from jax.experimental import pallas as pl
from jax.experimental.pallas import tpu as pltpu
from jax.experimental.pallas import tpu_sc as plsc

_NC = 2
_NS = 16


def _node_precompute(x, w_src, Wp, Wq, cf):
    n, d = x.shape

    def bf16_bits(f):
        ui = lax.bitcast_convert_type(f, jnp.int32)
        rnd = ui + 0x7FFF + (lax.shift_right_logical(ui, 16) & 1)
        return lax.shift_right_logical(rnd, 16)

    def pack_halves(t):
        lo = bf16_bits(t[:, :d // 2])
        hi = bf16_bits(t[:, d // 2:])
        return lo | lax.shift_left(hi, 16)

    def body(x_ref, w_ref, wp_ref, wq_ref, cf_ref, tpq_ref, u_ref):
        xv = x_ref[...]
        a = jnp.dot(xv, w_ref[...], preferred_element_type=jnp.float32)
        u = jnp.exp(a - jnp.max(a))
        alpha = jax.nn.sigmoid(cf_ref[...])
        xb = xv.astype(jnp.bfloat16)
        p = jnp.dot(xb, wp_ref[...].astype(jnp.bfloat16),
                    preferred_element_type=jnp.float32)
        q = jnp.dot(xb, wq_ref[...].astype(jnp.bfloat16),
                    preferred_element_type=jnp.float32)
        tpq_ref[:, :d // 2] = pack_halves((alpha * u) * p)
        tpq_ref[:, d // 2:] = pack_halves(((1.0 - alpha) * u) * q)
        u_ref[...] = u

    return pl.pallas_call(
        body,
        out_shape=[
            jax.ShapeDtypeStruct((n, d), jnp.int32),
            jax.ShapeDtypeStruct((n, 1), jnp.float32),
        ],
    )(x, w_src, Wp, Wq, cf)


def _edge_pass(tpq, u, src, dst, rf, zrows, zs):
    n, d = tpq.shape
    e = src.shape[0]
    nt = _NC * _NS
    ept = e // nt
    ch = 64
    npairs = ept // (2 * ch)
    main = npairs * 2 * ch
    tail = ept - main
    hg = d // 16
    rpt = (n // _NS) & ~7
    rrem = n - _NS * rpt

    mesh = plsc.VectorSubcoreMesh(
        core_axis_name="c", subcore_axis_name="s",
        num_cores=_NC, num_subcores=_NS)

    @functools.partial(
        pl.kernel,
        mesh=mesh,
        compiler_params=pltpu.CompilerParams(needs_layout_passes=False),
        out_type=[
            jax.ShapeDtypeStruct((_NC, n, d), jnp.float32),
            jax.ShapeDtypeStruct((nt, 5, 1, n // 5), jnp.float32),
        ],
        scratch_types=[
            pltpu.VMEM((n,), jnp.float32),
            pltpu.VMEM((n,), jnp.float32),
            pltpu.VMEM((ch,), jnp.int32),
            pltpu.VMEM((ch,), jnp.int32),
            pltpu.VMEM((ch,), jnp.float32),
            pltpu.VMEM((ch,), jnp.int32),
            pltpu.VMEM((ch,), jnp.float32),
            pltpu.VMEM((ch,), jnp.int32),
            pltpu.VMEM((ch,), jnp.int32),
            pltpu.VMEM((ch,), jnp.float32),
            pltpu.VMEM((ch,), jnp.int32),
            pltpu.VMEM((ch,), jnp.float32),
            pltpu.VMEM((ch, d), jnp.int32),
            pltpu.VMEM((ch, d), jnp.int32),
            pltpu.VMEM((ch, d), jnp.float32),
            pltpu.VMEM((16,), jnp.int32),
            pltpu.VMEM((16,), jnp.int32),
            pltpu.VMEM((16,), jnp.float32),
            pltpu.VMEM_SHARED((n, d), jnp.float32),
            pltpu.SemaphoreType.DMA,
            pltpu.SemaphoreType.DMA,
            pltpu.SemaphoreType.DMA,
            pltpu.SemaphoreType.DMA,
            pltpu.SemaphoreType.DMA,
        ],
    )
    def k(tpq_hbm, u_hbm, src_hbm, dst_hbm, rf_hbm, zrows_hbm, zs_hbm,
          acc_out, s_out,
          u_v, s_v,
          src_a, dst_a, rf_a, dsts_a, rfs_a,
          src_b, dst_b, rf_b, dsts_b, rfs_b,
          pq_a, pq_b, m,
          src_t, dst_t, rf_t,
          acc_sh,
          sem_ia, sem_ib, sem_pga, sem_pgb, sem_ps):
        c = lax.axis_index("c")
        s = lax.axis_index("s")
        wid = c * _NS + s
        tb = wid * ept

        @pl.when(s == 0)
        def _():
            pltpu.sync_copy(zrows_hbm, acc_sh)
        pltpu.sync_copy(zs_hbm, s_v)
        pltpu.sync_copy(u_hbm, u_v)
        plsc.subcore_barrier()

        zero16 = jnp.zeros((16,), jnp.int32)

        def s_channel(src_x, dst_x, nvec):
            for g in range(nvec):
                srcv = src_x[pl.ds(g * 16, 16)]
                dstv = dst_x[pl.ds(g * 16, 16)]
                uv = plsc.load_gather(u_v, [srcv])
                plsc.addupdate_scatter(s_v, [dstv], uv)

        def snapshot(from_x, to_x, nvec):
            for g in range(nvec):
                to_x[pl.ds(g * 16, 16)] = from_x[pl.ds(g * 16, 16)]

        himask = jnp.full((16,), -65536, jnp.int32)

        def scale(pq_x, m_x, rf_x, count):
            @plsc.parallel_loop(0, count, unroll=2)
            def _(ee):
                rfb = plsc.load_gather(rf_x, [zero16 + ee])
                for h in range(hg // 2):
                    pw = pq_x[ee, pl.ds(h * 16, 16)]
                    qw = pq_x[ee, pl.ds(d // 2 + h * 16, 16)]
                    plo = plsc.bitcast(lax.shift_left(pw, 16), jnp.float32)
                    phi = plsc.bitcast(pw & himask, jnp.float32)
                    qlo = plsc.bitcast(lax.shift_left(qw, 16), jnp.float32)
                    qhi = plsc.bitcast(qw & himask, jnp.float32)
                    m_x[ee, pl.ds(h * 32, 16)] = rfb * plo + qlo
                    m_x[ee, pl.ds(h * 32 + 16, 16)] = rfb * phi + qhi

        pltpu.sync_copy(src_hbm.at[pl.ds(tb + main, tail)], src_t)
        pltpu.sync_copy(dst_hbm.at[pl.ds(tb + main, tail)], dst_t)
        pltpu.sync_copy(rf_hbm.at[pl.ds(tb + main, tail)], rf_t)
        tg_cp = pltpu.async_copy(tpq_hbm.at[src_t], pq_a.at[pl.ds(0, tail)],
                                 sem_pga)
        s_channel(src_t, dst_t, tail // 16)
        tg_cp.wait()
        scale(pq_a, m, rf_t, tail)
        pltpu.sync_copy(m.at[pl.ds(0, tail)], acc_sh.at[dst_t], add=True)

        pltpu.sync_copy(src_hbm.at[pl.ds(tb, ch)], src_a)
        pltpu.sync_copy(dst_hbm.at[pl.ds(tb, ch)], dst_a)
        pltpu.sync_copy(rf_hbm.at[pl.ds(tb, ch)], rf_a)
        pltpu.async_copy(tpq_hbm.at[src_a], pq_a, sem_pga)
        pltpu.async_copy(src_hbm.at[pl.ds(tb + ch, ch)], src_b, sem_ib)
        pltpu.async_copy(dst_hbm.at[pl.ds(tb + ch, ch)], dst_b, sem_ib)
        pltpu.async_copy(rf_hbm.at[pl.ds(tb + ch, ch)], rf_b, sem_ib)

        def pair_body(j, carry):
            base = tb + j * (2 * ch)

            pltpu.make_async_copy(
                src_hbm.at[pl.ds(base + ch, ch)], src_b, sem_ib).wait()
            pltpu.make_async_copy(
                dst_hbm.at[pl.ds(base + ch, ch)], dst_b, sem_ib).wait()
            pltpu.make_async_copy(
                rf_hbm.at[pl.ds(base + ch, ch)], rf_b, sem_ib).wait()

            pltpu.async_copy(tpq_hbm.at[src_b], pq_b, sem_pgb)

            s_channel(src_a, dst_a, ch // 16)
            pltpu.make_async_copy(tpq_hbm.at[src_a], pq_a, sem_pga).wait()
            snapshot(dst_a, dsts_a, ch // 16)
            snapshot(rf_a, rfs_a, ch // 16)

            @pl.when(j < npairs - 1)
            def _():
                nbase = base + 2 * ch
                pltpu.async_copy(src_hbm.at[pl.ds(nbase, ch)], src_a, sem_ia)
                pltpu.async_copy(dst_hbm.at[pl.ds(nbase, ch)], dst_a, sem_ia)
                pltpu.async_copy(rf_hbm.at[pl.ds(nbase, ch)], rf_a, sem_ia)

            @pl.when(j > 0)
            def _():
                pltpu.make_async_copy(m, acc_sh.at[dsts_b], sem_ps).wait()

            scale(pq_a, m, rfs_a, ch)
            pltpu.async_copy(m, acc_sh.at[dsts_a], sem_ps, add=True)

            @pl.when(j < npairs - 1)
            def _():
                nbase = base + 2 * ch
                pltpu.make_async_copy(
                    src_hbm.at[pl.ds(nbase, ch)], src_a, sem_ia).wait()
                pltpu.make_async_copy(
                    dst_hbm.at[pl.ds(nbase, ch)], dst_a, sem_ia).wait()
                pltpu.make_async_copy(
                    rf_hbm.at[pl.ds(nbase, ch)], rf_a, sem_ia).wait()
                pltpu.async_copy(tpq_hbm.at[src_a], pq_a, sem_pga)

            s_channel(src_b, dst_b, ch // 16)
            pltpu.make_async_copy(tpq_hbm.at[src_b], pq_b, sem_pgb).wait()
            snapshot(dst_b, dsts_b, ch // 16)
            snapshot(rf_b, rfs_b, ch // 16)

            @pl.when(j < npairs - 1)
            def _():
                nb2 = base + 3 * ch
                pltpu.async_copy(src_hbm.at[pl.ds(nb2, ch)], src_b, sem_ib)
                pltpu.async_copy(dst_hbm.at[pl.ds(nb2, ch)], dst_b, sem_ib)
                pltpu.async_copy(rf_hbm.at[pl.ds(nb2, ch)], rf_b, sem_ib)

            pltpu.make_async_copy(m, acc_sh.at[dsts_a], sem_ps).wait()
            scale(pq_b, m, rfs_b, ch)
            pltpu.async_copy(m, acc_sh.at[dsts_b], sem_ps, add=True)
            return carry

        lax.fori_loop(0, npairs, pair_body, 0)

        pltpu.make_async_copy(m, acc_sh.at[dsts_b], sem_ps).wait()
        plsc.subcore_barrier()

        r0 = s * rpt
        pltpu.sync_copy(acc_sh.at[pl.ds(r0, rpt)],
                        acc_out.at[c, pl.ds(r0, rpt)])

        @pl.when(s == _NS - 1)
        def _():
            pltpu.sync_copy(acc_sh.at[pl.ds(_NS * rpt, rrem)],
                            acc_out.at[c, pl.ds(_NS * rpt, rrem)])

        for i in range(5):
            pltpu.sync_copy(s_v.at[pl.ds(i * (n // 5), n // 5)],
                            s_out.at[wid, i, 0])

    return k(tpq, u, src, dst, rf, zrows, zs)


def _gelu(v):
    return 0.5 * v * (1.0 + lax.erf(v * 0.7071067811865476))


def _ln(v, g, b, eps=1e-5):
    mu = jnp.mean(v, axis=-1, keepdims=True)
    var = jnp.mean((v - mu) ** 2, axis=-1, keepdims=True)
    return (v - mu) / jnp.sqrt(var + eps) * g + b


def _finish(acc2, s32, x, W1, b1, W2, b2, g1, beta1, g2, beta2):
    n, d = x.shape
    dh = W1.shape[1]
    nt = s32.shape[0]
    br = n // 5
    grid = n // br

    def body(acc_ref, s_ref, x_ref, w1_ref, b1_ref, w2_ref, b2_ref,
             g1_ref, be1_ref, g2_ref, be2_ref, o_ref):
        ssum = jnp.sum(s_ref[...].reshape(nt, br), axis=0)
        acc = acc_ref[0] + acc_ref[1]
        recip = jnp.where(ssum > 0, 1.0 / ssum, 0.0)
        msg = acc * recip[:, None]
        y = _gelu(msg) + x_ref[...]
        o1 = _ln(y, g1_ref[...], be1_ref[...])
        h1 = _gelu(jnp.dot(o1, w1_ref[...],
                           preferred_element_type=jnp.float32) + b1_ref[...])
        h = jnp.dot(h1, w2_ref[...],
                    preferred_element_type=jnp.float32) + b2_ref[...]
        o_ref[...] = _ln(h + o1, g2_ref[...], be2_ref[...])

    return pl.pallas_call(
        body,
        grid=(grid,),
        in_specs=[
            pl.BlockSpec((2, br, d), lambda i: (0, i, 0)),
            pl.BlockSpec((nt, 1, 1, br), lambda i: (0, i, 0, 0)),
            pl.BlockSpec((br, d), lambda i: (i, 0)),
            pl.BlockSpec((d, dh), lambda i: (0, 0)),
            pl.BlockSpec((dh,), lambda i: (0,)),
            pl.BlockSpec((dh, d), lambda i: (0, 0)),
            pl.BlockSpec((d,), lambda i: (0,)),
            pl.BlockSpec((d,), lambda i: (0,)),
            pl.BlockSpec((d,), lambda i: (0,)),
            pl.BlockSpec((d,), lambda i: (0,)),
            pl.BlockSpec((d,), lambda i: (0,)),
        ],
        out_specs=pl.BlockSpec((br, d), lambda i: (i, 0)),
        out_shape=jax.ShapeDtypeStruct((n, d), jnp.float32),
    )(acc2, s32, x, W1, b1, W2, b2, g1, beta1, g2, beta2)


def kernel(x, edge_index, W_phys, W_neur, att_w, channel_fusion, routing_factor,
           W1, b1, W2, b2, g1, beta1, g2, beta2):
    n, d = x.shape
    w_src = att_w[:d].reshape(d, 1)
    cf = jnp.asarray(channel_fusion, jnp.float32).reshape(1, 1)
    perm = np.empty((d,), np.int32)
    for p_ in range(d // 2):
        h2, i = divmod(p_, 16)
        perm[p_] = h2 * 32 + i
        perm[d // 2 + p_] = h2 * 32 + 16 + i
    tpq, u = _node_precompute(x, w_src, W_phys[:, perm], W_neur[:, perm], cf)
    src = edge_index[0]
    dst = edge_index[1]
    zrows = jnp.zeros((n, d), jnp.float32)
    zs = jnp.zeros((n,), jnp.float32)
    acc2, s32 = _edge_pass(tpq, u.reshape(n), src, dst, routing_factor,
                           zrows, zs)
    return _finish(acc2, s32, x, W1, b1, W2, b2, g1, beta1, g2, beta2)
